# Initial kernel scaffold; baseline (speedup 1.0000x reference)
import jax
import jax.numpy as jnp
from jax.experimental import pallas as pl


def kernel(z, codebook):
    raise NotImplementedError("write your pallas kernel here")


# speedup vs baseline: 1.7282x; 1.7282x over previous
import functools

import jax
import jax.numpy as jnp
from jax import lax
from jax.experimental import pallas as pl
from jax.experimental.pallas import tpu as pltpu
from jax.experimental.pallas import tpu_sc as plsc

_TOKENS = 8192
_K = 8192
_D = 32
_TBLK = 1024


def _argmin_body(z_ref, cbt_ref, idx_ref):
    z_blk = z_ref[...]
    cbt = cbt_ref[...]
    z2 = jnp.sum(z_blk * z_blk, axis=1, keepdims=True)
    cb2 = jnp.sum(cbt * cbt, axis=0, keepdims=True)
    dot = jnp.dot(z_blk, cbt, preferred_element_type=jnp.float32)
    d = (z2 + cb2) - 2.0 * dot
    idx_ref[...] = jnp.argmin(d, axis=1, keepdims=True).astype(jnp.int32)


def _compute_indices(z2d, cbt, interpret=False):
    return pl.pallas_call(
        _argmin_body,
        grid=(_TOKENS // _TBLK,),
        in_specs=[
            pl.BlockSpec((_TBLK, _D), lambda i: (i, 0)),
            pl.BlockSpec((_D, _K), lambda i: (0, 0)),
        ],
        out_specs=pl.BlockSpec((_TBLK, 1), lambda i: (i, 0)),
        out_shape=jax.ShapeDtypeStruct((_TOKENS, 1), jnp.int32),
        interpret=interpret,
    )(z2d, cbt)


_DPAD = 128


def _sc_gather(table_pad, idx):
    info = plsc.get_sparse_core_info()
    nc, ns = info.num_cores, info.num_subcores
    nw = nc * ns
    bpw = _TOKENS // nw
    mesh = plsc.VectorSubcoreMesh(core_axis_name="c", subcore_axis_name="s")

    @functools.partial(
        pl.kernel,
        mesh=mesh,
        out_type=jax.ShapeDtypeStruct((_TOKENS, _DPAD), jnp.float32),
        scratch_types=[
            pltpu.VMEM((bpw,), jnp.int32),
            pltpu.VMEM((bpw, _DPAD), jnp.float32),
            pltpu.SemaphoreType.DMA,
        ],
    )
    def gather(table_hbm, idx_hbm, out_hbm, idx_v, rows_v, sem):
        wid = lax.axis_index("s") * nc + lax.axis_index("c")
        base = wid * bpw
        pltpu.sync_copy(idx_hbm.at[pl.ds(base, bpw)], idx_v)
        pltpu.async_copy(table_hbm.at[idx_v], rows_v, sem).wait()
        pltpu.sync_copy(rows_v, out_hbm.at[pl.ds(base, bpw)])

    return gather(table_pad, idx)


def kernel(z, codebook):
    z2d = z.reshape(_TOKENS, _D)
    cbt = codebook.T
    idx = _compute_indices(z2d, cbt).reshape(_TOKENS)
    table_pad = jnp.pad(codebook, ((0, 0), (0, _DPAD - _D)))
    x_q = _sc_gather(table_pad, idx)[:, :_D]
    return x_q.reshape(z.shape)


# speedup vs baseline: 1.8627x; 1.0779x over previous
import functools

import jax
import jax.numpy as jnp
from jax import lax
from jax.experimental import pallas as pl
from jax.experimental.pallas import tpu as pltpu
from jax.experimental.pallas import tpu_sc as plsc

_TOKENS = 8192
_K = 8192
_D = 32
_TBLK = 1024


def _argmin_body(z_ref, cbt_ref, idx_ref):
    z_blk = z_ref[...]
    cbt = cbt_ref[...]
    z2 = jnp.sum(z_blk * z_blk, axis=1, keepdims=True)
    cb2 = jnp.sum(cbt * cbt, axis=0, keepdims=True)
    dot2 = jnp.dot(z_blk + z_blk, cbt, preferred_element_type=jnp.float32)
    d = (z2 + cb2) - dot2
    idx_ref[...] = jnp.argmin(d, axis=1, keepdims=True).astype(jnp.int32)


def _compute_indices(z2d, cbt, interpret=False):
    return pl.pallas_call(
        _argmin_body,
        grid=(_TOKENS // _TBLK,),
        in_specs=[
            pl.BlockSpec((_TBLK, _D), lambda i: (i, 0)),
            pl.BlockSpec((_D, _K), lambda i: (0, 0)),
        ],
        out_specs=pl.BlockSpec((_TBLK, 1), lambda i: (i, 0)),
        out_shape=jax.ShapeDtypeStruct((_TOKENS, 1), jnp.int32),
        interpret=interpret,
    )(z2d, cbt)


def _sc_gather(codebook, idx):
    info = plsc.get_sparse_core_info()
    nc, ns = info.num_cores, info.num_subcores
    nw = nc * ns
    bpw = _TOKENS // nw
    mesh = plsc.VectorSubcoreMesh(core_axis_name="c", subcore_axis_name="s")

    @functools.partial(
        pl.kernel,
        mesh=mesh,
        out_type=jax.ShapeDtypeStruct((_TOKENS, _D), jnp.float32),
        scratch_types=[
            pltpu.VMEM((bpw,), jnp.int32),
            pltpu.VMEM((bpw, _D), jnp.float32),
            pltpu.SemaphoreType.DMA,
        ],
        compiler_params=pltpu.CompilerParams(use_tc_tiling_on_sc=False),
    )
    def gather(table_hbm, idx_hbm, out_hbm, idx_v, rows_v, sem):
        wid = lax.axis_index("s") * nc + lax.axis_index("c")
        base = wid * bpw
        pltpu.sync_copy(idx_hbm.at[pl.ds(base, bpw)], idx_v)
        pltpu.async_copy(table_hbm.at[idx_v], rows_v, sem).wait()
        pltpu.sync_copy(rows_v, out_hbm.at[pl.ds(base, bpw)])

    return gather(codebook, idx)


def kernel(z, codebook):
    z2d = z.reshape(_TOKENS, _D)
    cbt = codebook.T
    idx = _compute_indices(z2d, cbt).reshape(_TOKENS)
    x_q = _sc_gather(codebook, idx)
    return x_q.reshape(z.shape)
